# baseline (speedup 1.0000x reference)
import functools

import jax
import jax.numpy as jnp
from jax import lax
from jax.experimental import pallas as pl
from jax.experimental.pallas import tpu as pltpu
from jax.experimental.pallas import tpu_sc as plsc

D = 768
DFF = 4 * D
E = 4
K = 2
ALT = 32
GH = D // 2
NEG_INF = float("-inf")

BN = 4096
NP = 2 * BN
TILE = 256
SPAD = NP + E * TILE
NT = SPAD // TILE
TSZ = 48
DW = D // 2

NC = 2
NS = 16
NWORK = NC * NS


def _gelu_exact(x):
    return 0.5 * x * (1.0 + jax.lax.erf(x * (2.0 ** -0.5)))


def _router_body(nb, n_per_b, tok_ref, alt_ref, lnw_ref, lnb_ref,
                 gw1d_ref, gw1a_ref, gb1_ref, gw2_ref, gb2_ref,
                 tnbf_ref, e0_ref, e1_ref, g0_ref, g1_ref, lb_ref,
                 f_acc, p_acc):
    i = pl.program_id(0)
    x = tok_ref[...]
    mu = jnp.mean(x, axis=1, keepdims=True)
    xc = x - mu
    var = jnp.mean(xc * xc, axis=1, keepdims=True)
    tn = xc * jax.lax.rsqrt(var + 1e-5) * lnw_ref[...] + lnb_ref[...]
    tnbf_ref[...] = tn.astype(jnp.bfloat16)

    alt_c = jnp.dot(alt_ref[...], gw1a_ref[...],
                    preferred_element_type=jnp.float32)
    b = i // n_per_b
    sel = jax.lax.broadcasted_iota(jnp.int32, alt_c.shape, 0) == b
    ac = jnp.sum(jnp.where(sel, alt_c, 0.0), axis=0, keepdims=True)

    h_pre = jnp.dot(tn, gw1d_ref[...],
                    preferred_element_type=jnp.float32) + ac + gb1_ref[...]
    h = _gelu_exact(h_pre)
    logits = jnp.dot(h, gw2_ref[...],
                     preferred_element_type=jnp.float32) + gb2_ref[...]

    iota_e = jax.lax.broadcasted_iota(jnp.int32, logits.shape, 1)
    m0 = jnp.max(logits, axis=1, keepdims=True)
    e0 = jnp.min(jnp.where(logits == m0, iota_e, E), axis=1, keepdims=True)
    masked = jnp.where(iota_e == e0, NEG_INF, logits)
    m1 = jnp.max(masked, axis=1, keepdims=True)
    e1 = jnp.min(jnp.where(masked == m1, iota_e, E), axis=1, keepdims=True)

    z = jnp.exp(m1 - m0)
    g0 = 1.0 / (1.0 + z)
    g1 = z / (1.0 + z)

    p = jnp.exp(logits - m0)
    p = p / jnp.sum(p, axis=1, keepdims=True)

    e0_ref[...] = e0
    e1_ref[...] = e1
    g0_ref[...] = g0
    g1_ref[...] = g1

    f_part = jnp.sum((iota_e == e0).astype(jnp.float32), axis=0, keepdims=True)
    p_part = jnp.sum(p, axis=0, keepdims=True)

    @pl.when(i == 0)
    def _():
        f_acc[...] = f_part
        p_acc[...] = p_part

    @pl.when(i > 0)
    def _():
        f_acc[...] += f_part
        p_acc[...] += p_part

    @pl.when(i == nb - 1)
    def _():
        bn2 = float((nb * x.shape[0]) ** 2)
        lb_ref[...] = (E / bn2) * jnp.sum(f_acc[...] * p_acc[...],
                                          axis=1, keepdims=True)


def _run_router(tok2d, alt, lnw, lnb, gw1d, gw1a, gb1, gw2, gb2, n):
    bn = tok2d.shape[0]
    bt = 512
    nb = bn // bt
    n_per_b = n // bt
    body = functools.partial(_router_body, nb, n_per_b)
    return pl.pallas_call(
        body,
        grid=(nb,),
        in_specs=[
            pl.BlockSpec((bt, D), lambda i: (i, 0)),
            pl.BlockSpec(alt.shape, lambda i: (0, 0)),
            pl.BlockSpec((1, D), lambda i: (0, 0)),
            pl.BlockSpec((1, D), lambda i: (0, 0)),
            pl.BlockSpec((D, GH), lambda i: (0, 0)),
            pl.BlockSpec((ALT, GH), lambda i: (0, 0)),
            pl.BlockSpec((1, GH), lambda i: (0, 0)),
            pl.BlockSpec((GH, E), lambda i: (0, 0)),
            pl.BlockSpec((1, E), lambda i: (0, 0)),
        ],
        out_specs=[
            pl.BlockSpec((bt, D), lambda i: (i, 0)),
            pl.BlockSpec((bt, 1), lambda i: (i, 0)),
            pl.BlockSpec((bt, 1), lambda i: (i, 0)),
            pl.BlockSpec((bt, 1), lambda i: (i, 0)),
            pl.BlockSpec((bt, 1), lambda i: (i, 0)),
            pl.BlockSpec((1, 1), lambda i: (0, 0)),
        ],
        out_shape=[
            jax.ShapeDtypeStruct((bn, D), jnp.bfloat16),
            jax.ShapeDtypeStruct((bn, 1), jnp.int32),
            jax.ShapeDtypeStruct((bn, 1), jnp.int32),
            jax.ShapeDtypeStruct((bn, 1), jnp.float32),
            jax.ShapeDtypeStruct((bn, 1), jnp.float32),
            jax.ShapeDtypeStruct((1, 1), jnp.float32),
        ],
        scratch_shapes=[
            pltpu.VMEM((1, E), jnp.float32),
            pltpu.VMEM((1, E), jnp.float32),
        ],
    )(tok2d, alt, lnw, lnb, gw1d, gw1a, gb1, gw2, gb2)


def _temp_sort(e0, e1):
    e_all = jnp.concatenate([e0, e1])
    order = jnp.argsort(e_all, stable=True)
    se = e_all[order]
    counts = jnp.sum(jax.nn.one_hot(e_all, E, dtype=jnp.int32), axis=0)
    padded = ((counts + TILE - 1) // TILE) * TILE
    base = jnp.concatenate(
        [jnp.zeros((1,), jnp.int32), jnp.cumsum(padded)[:-1]])
    startc = jnp.concatenate(
        [jnp.zeros((1,), jnp.int32), jnp.cumsum(counts)[:-1]])
    q = base[se] + (jnp.arange(NP, dtype=jnp.int32) - startc[se])
    ps = jnp.zeros((NP,), jnp.int32).at[order].set(q)
    st = jnp.zeros((SPAD,), jnp.int32).at[q].set(order % BN)
    sp = jnp.zeros((SPAD,), jnp.int32).at[q].set(order)
    rowstart = jnp.arange(TSZ, dtype=jnp.int32) * TILE
    p1 = padded[0]
    p2 = p1 + padded[1]
    p3 = p2 + padded[2]
    te = ((rowstart >= p1).astype(jnp.int32)
          + (rowstart >= p2).astype(jnp.int32)
          + (rowstart >= p3).astype(jnp.int32))
    return st, sp, ps, te


def _gather_body(tn_hbm, st_hbm, sp_hbm, g_hbm, xs_hbm, gs_hbm,
                 idx_v, rows_v, gs_v, sem):
    wid = lax.axis_index("s") * NC + lax.axis_index("c")
    rpw = SPAD // NWORK
    base = wid * rpw
    pltpu.sync_copy(st_hbm.at[pl.ds(base, rpw)], idx_v)
    pltpu.async_copy(tn_hbm.at[idx_v], rows_v, sem).wait()
    pltpu.sync_copy(rows_v, xs_hbm.at[pl.ds(base, rpw)])
    pltpu.sync_copy(sp_hbm.at[pl.ds(base, rpw)], idx_v)
    pltpu.async_copy(g_hbm.at[idx_v], gs_v, sem).wait()
    pltpu.sync_copy(gs_v, gs_hbm.at[pl.ds(base, rpw)])


def _run_gather(tn_i32, st, sp, g_all):
    rpw = SPAD // NWORK
    mesh = plsc.VectorSubcoreMesh(core_axis_name="c", subcore_axis_name="s")
    f = functools.partial(
        pl.kernel, mesh=mesh,
        out_type=[
            jax.ShapeDtypeStruct((SPAD, DW), jnp.int32),
            jax.ShapeDtypeStruct((SPAD,), jnp.float32),
        ],
        scratch_types=[
            pltpu.VMEM((rpw,), jnp.int32),
            pltpu.VMEM((rpw, DW), jnp.int32),
            pltpu.VMEM((rpw,), jnp.float32),
            pltpu.SemaphoreType.DMA,
        ],
    )(_gather_body)
    return f(tn_i32, st, sp, g_all)


DFFB = 1536
NJ = DFF // DFFB


def _gffn_body(te_ref, xs_ref, gs_ref, w1_ref, b1_ref, w2_ref, b2_ref,
               ys_ref):
    j = pl.program_id(0)
    t = pl.program_id(1)
    rows = pl.ds(t * TILE, TILE)
    x = xs_ref[...]
    gs = gs_ref[...]
    w1 = w1_ref[0].astype(jnp.bfloat16)
    w2 = w2_ref[0].astype(jnp.bfloat16)
    h = jnp.dot(x, w1, preferred_element_type=jnp.float32)
    h = _gelu_exact(h + b1_ref[0])
    y = jnp.dot(h.astype(jnp.bfloat16), w2,
                preferred_element_type=jnp.float32)

    @pl.when(j == 0)
    def _():
        ys_ref[rows, :] = (y + b2_ref[0]) * gs

    @pl.when(j > 0)
    def _():
        ys_ref[rows, :] += y * gs


def _run_gffn(te, xs, gs, w1, b1, w2, b2):
    grid_spec = pltpu.PrefetchScalarGridSpec(
        num_scalar_prefetch=1,
        grid=(NJ, NT),
        in_specs=[
            pl.BlockSpec((TILE, D), lambda j, t, te: (t, 0)),
            pl.BlockSpec((TILE, 1), lambda j, t, te: (t, 0)),
            pl.BlockSpec((1, D, DFFB), lambda j, t, te: (te[t], 0, j)),
            pl.BlockSpec((1, 1, DFFB), lambda j, t, te: (te[t], 0, j)),
            pl.BlockSpec((1, DFFB, D), lambda j, t, te: (te[t], j, 0)),
            pl.BlockSpec((1, 1, D), lambda j, t, te: (te[t], 0, 0)),
        ],
        out_specs=pl.BlockSpec((SPAD, D), lambda j, t, te: (0, 0)),
    )
    return pl.pallas_call(
        _gffn_body,
        grid_spec=grid_spec,
        out_shape=jax.ShapeDtypeStruct((SPAD, D), jnp.float32),
    )(te, xs, gs, w1, b1, w2, b2)


CCH = 32


def _combine_body(ys_hbm, p0_hbm, p1_hbm, tok_hbm, out_hbm,
                  i0_v, i1_v, y0_v, y1_v, t_v, sem):
    wid = lax.axis_index("s") * NC + lax.axis_index("c")
    tpw = BN // NWORK

    def chunk_body(c, _):
        base = wid * tpw + c * CCH
        pltpu.sync_copy(p0_hbm.at[pl.ds(base, CCH)], i0_v)
        pltpu.sync_copy(p1_hbm.at[pl.ds(base, CCH)], i1_v)
        pltpu.sync_copy(tok_hbm.at[pl.ds(base, CCH)], t_v)
        pltpu.async_copy(ys_hbm.at[i0_v], y0_v, sem).wait()
        pltpu.async_copy(ys_hbm.at[i1_v], y1_v, sem).wait()

        def row_body(r, _):
            for cc in range(D // 16):
                cs = pl.ds(cc * 16, 16)
                t_v[r, cs] = t_v[r, cs] + y0_v[r, cs] + y1_v[r, cs]
            return 0

        lax.fori_loop(0, CCH, row_body, 0)
        pltpu.sync_copy(t_v, out_hbm.at[pl.ds(base, CCH)])
        return 0

    lax.fori_loop(0, tpw // CCH, chunk_body, 0)


def _run_combine(ys, p0, p1, tok2d):
    mesh = plsc.VectorSubcoreMesh(core_axis_name="c", subcore_axis_name="s")
    f = functools.partial(
        pl.kernel, mesh=mesh,
        out_type=jax.ShapeDtypeStruct((BN, D), jnp.float32),
        scratch_types=[
            pltpu.VMEM((CCH,), jnp.int32),
            pltpu.VMEM((CCH,), jnp.int32),
            pltpu.VMEM((CCH, D), jnp.float32),
            pltpu.VMEM((CCH, D), jnp.float32),
            pltpu.VMEM((CCH, D), jnp.float32),
            pltpu.SemaphoreType.DMA,
        ],
    )(_combine_body)
    return f(ys, p0, p1, tok2d)


def kernel(tokens, alt_embedding, ln_w, ln_b, gate_w1, gate_b1, gate_w2,
           gate_b2, exp_w1, exp_b1, exp_w2, exp_b2):
    b, n, d = tokens.shape
    bn = b * n
    tok2d = tokens.reshape(bn, d)
    gw1d = gate_w1[:d]
    gw1a = gate_w1[d:]

    tnbf, e0, e1, g0, g1, lb = _run_router(
        tok2d, alt_embedding, ln_w.reshape(1, d), ln_b.reshape(1, d),
        gw1d, gw1a, gate_b1.reshape(1, GH), gate_w2,
        gate_b2.reshape(1, E), n)

    st, sp, ps, te = _temp_sort(e0.reshape(bn), e1.reshape(bn))
    g_all = jnp.concatenate([g0.reshape(bn), g1.reshape(bn)])

    tn_i32 = jax.lax.bitcast_convert_type(
        tnbf.reshape(bn, DW, 2), jnp.int32)
    xs_i32, gs = _run_gather(tn_i32, st, sp, g_all)
    xs = jax.lax.bitcast_convert_type(
        xs_i32, jnp.bfloat16).reshape(SPAD, d)

    ys = _run_gffn(te, xs, gs.reshape(SPAD, 1), exp_w1,
                   exp_b1.reshape(E, 1, DFF), exp_w2, exp_b2.reshape(E, 1, D))

    out = _run_combine(ys, ps[:bn], ps[bn:], tok2d)

    return (out.reshape(b, n, d), lb[0, 0])


# speedup vs baseline: 2.7354x; 2.7354x over previous
import functools

import jax
import jax.numpy as jnp
from jax.experimental import pallas as pl
from jax.experimental.pallas import tpu as pltpu

D = 768
DFF = 4 * D
E = 4
K = 2
ALT = 32
GH = D // 2
NEG_INF = float("-inf")


def _gelu_exact(x):
    return 0.5 * x * (1.0 + jax.lax.erf(x * (2.0 ** -0.5)))


def _router_body(nb, n_per_b, tok_ref, alt_ref, lnw_ref, lnb_ref,
                 gw1d_ref, gw1a_ref, gb1_ref, gw2_ref, gb2_ref,
                 tn32_ref, tnbf_ref, wcomb_ref, lb_ref, f_acc, p_acc):
    i = pl.program_id(0)
    x = tok_ref[...]
    mu = jnp.mean(x, axis=1, keepdims=True)
    xc = x - mu
    var = jnp.mean(xc * xc, axis=1, keepdims=True)
    tn = xc * jax.lax.rsqrt(var + 1e-5) * lnw_ref[...] + lnb_ref[...]
    tn32_ref[...] = tn
    tnbf_ref[...] = tn.astype(jnp.bfloat16)

    alt_c = jnp.dot(alt_ref[...], gw1a_ref[...],
                    preferred_element_type=jnp.float32)
    b = i // n_per_b
    sel = jax.lax.broadcasted_iota(jnp.int32, alt_c.shape, 0) == b
    ac = jnp.sum(jnp.where(sel, alt_c, 0.0), axis=0, keepdims=True)

    h_pre = jnp.dot(tn, gw1d_ref[...],
                    preferred_element_type=jnp.float32) + ac + gb1_ref[...]
    h = _gelu_exact(h_pre)
    logits = jnp.dot(h, gw2_ref[...],
                     preferred_element_type=jnp.float32) + gb2_ref[...]

    iota_e = jax.lax.broadcasted_iota(jnp.int32, logits.shape, 1)
    m0 = jnp.max(logits, axis=1, keepdims=True)
    e0 = jnp.min(jnp.where(logits == m0, iota_e, E), axis=1, keepdims=True)
    masked = jnp.where(iota_e == e0, NEG_INF, logits)
    m1 = jnp.max(masked, axis=1, keepdims=True)
    e1 = jnp.min(jnp.where(masked == m1, iota_e, E), axis=1, keepdims=True)

    z = jnp.exp(m1 - m0)
    g0 = 1.0 / (1.0 + z)
    g1 = z / (1.0 + z)

    p = jnp.exp(logits - m0)
    p = p / jnp.sum(p, axis=1, keepdims=True)

    wcomb_ref[...] = (jnp.where(iota_e == e0, g0, 0.0)
                      + jnp.where(iota_e == e1, g1, 0.0))

    f_part = jnp.sum((iota_e == e0).astype(jnp.float32), axis=0, keepdims=True)
    p_part = jnp.sum(p, axis=0, keepdims=True)

    @pl.when(i == 0)
    def _():
        f_acc[...] = f_part
        p_acc[...] = p_part

    @pl.when(i > 0)
    def _():
        f_acc[...] += f_part
        p_acc[...] += p_part

    @pl.when(i == nb - 1)
    def _():
        bn2 = float((nb * x.shape[0]) ** 2)
        lb_ref[...] = (E / bn2) * jnp.sum(f_acc[...] * p_acc[...],
                                          axis=1, keepdims=True)


def _run_router(tok2d, alt, lnw, lnb, gw1d, gw1a, gb1, gw2, gb2, n):
    bn = tok2d.shape[0]
    bt = 512
    nb = bn // bt
    n_per_b = n // bt
    body = functools.partial(_router_body, nb, n_per_b)
    return pl.pallas_call(
        body,
        grid=(nb,),
        in_specs=[
            pl.BlockSpec((bt, D), lambda i: (i, 0)),
            pl.BlockSpec(alt.shape, lambda i: (0, 0)),
            pl.BlockSpec((1, D), lambda i: (0, 0)),
            pl.BlockSpec((1, D), lambda i: (0, 0)),
            pl.BlockSpec((D, GH), lambda i: (0, 0)),
            pl.BlockSpec((ALT, GH), lambda i: (0, 0)),
            pl.BlockSpec((1, GH), lambda i: (0, 0)),
            pl.BlockSpec((GH, E), lambda i: (0, 0)),
            pl.BlockSpec((1, E), lambda i: (0, 0)),
        ],
        out_specs=[
            pl.BlockSpec((bt, D), lambda i: (i, 0)),
            pl.BlockSpec((bt, D), lambda i: (i, 0)),
            pl.BlockSpec((bt, E), lambda i: (i, 0)),
            pl.BlockSpec((1, 1), lambda i: (0, 0)),
        ],
        out_shape=[
            jax.ShapeDtypeStruct((bn, D), jnp.float32),
            jax.ShapeDtypeStruct((bn, D), jnp.bfloat16),
            jax.ShapeDtypeStruct((bn, E), jnp.float32),
            jax.ShapeDtypeStruct((1, 1), jnp.float32),
        ],
        scratch_shapes=[
            pltpu.VMEM((1, E), jnp.float32),
            pltpu.VMEM((1, E), jnp.float32),
        ],
    )(tok2d, alt, lnw, lnb, gw1d, gw1a, gb1, gw2, gb2)


DFFB = 1536
NJ = DFF // DFFB
FBT = 1024
FNB = 4


def _dense_ffn_body(tnbf_ref, wcomb_ref, tok_ref, w1_ref, b1_ref,
                    w2_ref, b2_ref, out_ref):
    e = pl.program_id(0)
    j = pl.program_id(1)
    i = pl.program_id(2)
    rows = pl.ds(i * FBT, FBT)
    x = tnbf_ref[rows, :]
    w1 = w1_ref[0].astype(jnp.bfloat16)
    w2 = w2_ref[0].astype(jnp.bfloat16)
    h = jnp.dot(x, w1, preferred_element_type=jnp.float32)
    h = _gelu_exact(h + b1_ref[0])
    y = jnp.dot(h.astype(jnp.bfloat16), w2,
                preferred_element_type=jnp.float32)
    iota_e = jax.lax.broadcasted_iota(jnp.int32, (FBT, E), 1)
    w = jnp.sum(jnp.where(iota_e == e, wcomb_ref[rows, :], 0.0),
                axis=1, keepdims=True)

    @pl.when(j == 0)
    def _():
        y_b = y + b2_ref[0]

        @pl.when(e == 0)
        def _():
            out_ref[rows, :] = tok_ref[rows, :] + w * y_b

        @pl.when(e > 0)
        def _():
            out_ref[rows, :] += w * y_b

    @pl.when(j > 0)
    def _():
        out_ref[rows, :] += w * y


def _run_dense_ffn(tnbf, wcomb, tok2d, w1, b1, w2, b2):
    bn = tnbf.shape[0]
    return pl.pallas_call(
        _dense_ffn_body,
        grid=(E, NJ, FNB),
        in_specs=[
            pl.BlockSpec((bn, D), lambda e, j, i: (0, 0)),
            pl.BlockSpec((bn, E), lambda e, j, i: (0, 0)),
            pl.BlockSpec((bn, D), lambda e, j, i: (0, 0)),
            pl.BlockSpec((1, D, DFFB), lambda e, j, i: (e, 0, j)),
            pl.BlockSpec((1, 1, DFFB), lambda e, j, i: (e, 0, j)),
            pl.BlockSpec((1, DFFB, D), lambda e, j, i: (e, j, 0)),
            pl.BlockSpec((1, 1, D), lambda e, j, i: (e, 0, 0)),
        ],
        out_specs=pl.BlockSpec((bn, D), lambda e, j, i: (0, 0)),
        out_shape=jax.ShapeDtypeStruct((bn, D), jnp.float32),
    )(tnbf, wcomb, tok2d, w1, b1, w2, b2)


def kernel(tokens, alt_embedding, ln_w, ln_b, gate_w1, gate_b1, gate_w2,
           gate_b2, exp_w1, exp_b1, exp_w2, exp_b2):
    b, n, d = tokens.shape
    bn = b * n
    tok2d = tokens.reshape(bn, d)
    gw1d = gate_w1[:d]
    gw1a = gate_w1[d:]

    tn32, tnbf, wcomb, lb = _run_router(
        tok2d, alt_embedding, ln_w.reshape(1, d), ln_b.reshape(1, d),
        gw1d, gw1a, gate_b1.reshape(1, GH), gate_w2,
        gate_b2.reshape(1, E), n)

    out = _run_dense_ffn(
        tnbf, wcomb, tok2d,
        exp_w1, exp_b1.reshape(E, 1, DFF),
        exp_w2, exp_b2.reshape(E, 1, D))

    return (out.reshape(b, n, d), lb[0, 0])
